# baseline (device time: 109191 ns/iter reference)
import functools

import jax
import jax.numpy as jnp
from jax import lax
from jax.experimental import pallas as pl
from jax.experimental.pallas import tpu as pltpu

N_DEV = 16
HEADS_PER = 8
SQ = 512
SKV = 2048
DH = 128
D_LOCAL = HEADS_PER * DH
D_MODEL = 1024
SCALE = 0.08838834764831843

ROWS_PER_SLOT = SQ // N_DEV

RS_MASKS = [1, 4, 2, 8]
B_LIST = [0, 2, 1, 3]
RS_RECV_OFF = [0, 8, 12, 14]


def _slot(c: int) -> int:
    b0, b1, b2, b3 = c & 1, (c >> 1) & 1, (c >> 2) & 1, (c >> 3) & 1
    return (b0 << 3) | (b2 << 2) | (b1 << 1) | b3


def kernel(x, Wq, Wo, K_ext, V_ext):
    me_outer = lax.axis_index("i")
    h0 = me_outer * HEADS_PER
    K_loc = lax.dynamic_slice_in_dim(K_ext[0], h0, HEADS_PER, axis=1)
    V_loc = lax.dynamic_slice_in_dim(V_ext[0], h0, HEADS_PER, axis=1)
    xb = x[0].astype(jnp.bfloat16)
    Wqb = Wq.astype(jnp.bfloat16)
    Wob = Wo.astype(jnp.bfloat16)
    Kb = K_loc.astype(jnp.bfloat16)
    Vb = V_loc.astype(jnp.bfloat16)

    def body(x_ref, wq_ref, wo_ref, k_ref, v_ref, out_ref,
             attn_ref, buf_ref, recv_ref, send_sems, recv_sems):
        me = lax.axis_index("i")

        q = jnp.dot(x_ref[...], wq_ref[...],
                    preferred_element_type=jnp.float32)
        qb = q.astype(jnp.bfloat16)
        for h in range(HEADS_PER):
            qh = qb[:, h * DH:(h + 1) * DH]
            kh = k_ref[:, h, :]
            vh = v_ref[:, h, :]
            s = lax.dot_general(
                qh, kh, (((1,), (1,)), ((), ())),
                preferred_element_type=jnp.float32) * SCALE
            m = jnp.max(s, axis=1, keepdims=True)
            p = jnp.exp(s - m)
            l = jnp.sum(p, axis=1, keepdims=True)
            oh = jnp.dot(p.astype(jnp.bfloat16), vh,
                         preferred_element_type=jnp.float32)
            attn_ref[:, h * DH:(h + 1) * DH] = (oh / l).astype(jnp.bfloat16)

        partial = jnp.dot(attn_ref[...], wo_ref[...],
                          preferred_element_type=jnp.float32)
        for c in range(N_DEV):
            sl = _slot(c)
            buf_ref[sl * ROWS_PER_SLOT:(sl + 1) * ROWS_PER_SLOT, :] = \
                partial[c * ROWS_PER_SLOT:(c + 1) * ROWS_PER_SLOT, :]

        barrier = pltpu.get_barrier_semaphore()
        for mm in RS_MASKS:
            pl.semaphore_signal(barrier, inc=1, device_id=(me ^ mm,),
                                device_id_type=pl.DeviceIdType.MESH)
        pl.semaphore_wait(barrier, len(RS_MASKS))

        lo = jnp.int32(0)
        for k in range(4):
            mask = RS_MASKS[k]
            half = 8 >> k
            partner = me ^ mask
            mybit = (me >> B_LIST[k]) & 1
            send_lo = lo + (1 - mybit) * half
            keep_lo = lo + mybit * half
            off = RS_RECV_OFF[k]
            rdma = pltpu.make_async_remote_copy(
                src_ref=buf_ref.at[pl.ds(send_lo * ROWS_PER_SLOT,
                                         half * ROWS_PER_SLOT)],
                dst_ref=recv_ref.at[pl.ds(off * ROWS_PER_SLOT,
                                          half * ROWS_PER_SLOT)],
                send_sem=send_sems.at[k],
                recv_sem=recv_sems.at[k],
                device_id=(partner,),
                device_id_type=pl.DeviceIdType.MESH,
            )
            rdma.start()
            rdma.wait()
            kls = pl.ds(keep_lo * ROWS_PER_SLOT, half * ROWS_PER_SLOT)
            buf_ref[kls, :] = buf_ref[kls, :] + recv_ref[
                off * ROWS_PER_SLOT:(off + half) * ROWS_PER_SLOT, :]
            lo = keep_lo

        ln = 1
        for k in range(4):
            mask = RS_MASKS[3 - k]
            partner = me ^ mask
            mybit = (me >> B_LIST[3 - k]) & 1
            rdma = pltpu.make_async_remote_copy(
                src_ref=buf_ref.at[pl.ds(lo * ROWS_PER_SLOT,
                                         ln * ROWS_PER_SLOT)],
                dst_ref=buf_ref.at[pl.ds(lo * ROWS_PER_SLOT,
                                         ln * ROWS_PER_SLOT)],
                send_sem=send_sems.at[4 + k],
                recv_sem=recv_sems.at[4 + k],
                device_id=(partner,),
                device_id_type=pl.DeviceIdType.MESH,
            )
            rdma.start()
            rdma.wait()
            lo = lo - mybit * ln
            ln = ln * 2

        for c in range(N_DEV):
            sl = _slot(c)
            out_ref[c * ROWS_PER_SLOT:(c + 1) * ROWS_PER_SLOT, :] = \
                buf_ref[sl * ROWS_PER_SLOT:(sl + 1) * ROWS_PER_SLOT, :]

        @functools.partial(pl.run_scoped,
                           second_barrier=pltpu.SemaphoreType.REGULAR)
        def _(second_barrier):
            for mm in RS_MASKS:
                pl.semaphore_signal(second_barrier, inc=1,
                                    device_id=(me ^ mm,),
                                    device_id_type=pl.DeviceIdType.MESH)
            pl.semaphore_wait(second_barrier, len(RS_MASKS))

    out = pl.pallas_call(
        body,
        out_shape=jax.ShapeDtypeStruct((SQ, D_MODEL), jnp.float32),
        in_specs=[pl.BlockSpec(memory_space=pltpu.VMEM)] * 5,
        out_specs=pl.BlockSpec(memory_space=pltpu.VMEM),
        scratch_shapes=[
            pltpu.VMEM((SQ, D_LOCAL), jnp.bfloat16),
            pltpu.VMEM((SQ, D_MODEL), jnp.float32),
            pltpu.VMEM((15 * ROWS_PER_SLOT, D_MODEL), jnp.float32),
            pltpu.SemaphoreType.DMA((8,)),
            pltpu.SemaphoreType.DMA((8,)),
        ],
        compiler_params=pltpu.CompilerParams(collective_id=0),
    )(xb, Wqb, Wob, Kb, Vb)
    return out[None]
